# baseline (speedup 1.0000x reference)
import jax
import jax.numpy as jnp
from jax.experimental import pallas as pl
from jax.experimental.pallas import tpu as pltpu

_H, _W = 28, 28


def _upsample_kernel(x_ref, w_ref, b_ref, o_ref):
    TR, _, W2, C_out = o_ref.shape
    W = W2 // 2
    n2 = 2 * C_out
    xb = x_ref[...].astype(jnp.bfloat16)
    acc = jnp.dot(xb, w_ref[...], preferred_element_type=jnp.float32)
    acc = acc + b_ref[...]
    o_ref[:, 0, :, :] = acc[:, :n2].reshape(TR * W * 2, C_out).reshape(
        TR, W2, C_out)
    o_ref[:, 1, :, :] = acc[:, n2:].reshape(TR * W * 2, C_out).reshape(
        TR, W2, C_out)


def kernel(x, weight, bias):
    H, W = _H, _W
    B, L, C_in = x.shape
    assert L == H * W
    _, C_out, kh_, kw_ = weight.shape
    assert (kh_, kw_) == (2, 2)

    N = 4 * C_out
    BH = B * H

    x2d = x.reshape(BH * W, C_in)
    w_mat = jnp.transpose(weight, (0, 2, 3, 1)).reshape(C_in, N)
    w_mat = w_mat.astype(jnp.bfloat16)
    b_mat = jnp.tile(bias.astype(jnp.float32), 4).reshape(1, N)

    TR = 224
    if BH % TR != 0:
        TR = 8 if BH % 8 == 0 else 1
    grid = (BH // TR,)

    cost = pl.CostEstimate(
        flops=2 * BH * W * C_in * N,
        bytes_accessed=(x2d.size * 4 + w_mat.size * 2 + b_mat.size * 4
                        + BH * 4 * W * C_out * 4),
        transcendentals=0,
    )

    out4d = pl.pallas_call(
        _upsample_kernel,
        out_shape=jax.ShapeDtypeStruct((BH, 2, 2 * W, C_out), x.dtype),
        grid=grid,
        in_specs=[
            pl.BlockSpec((TR * W, C_in), lambda i: (i, 0)),
            pl.BlockSpec((C_in, N), lambda i: (0, 0)),
            pl.BlockSpec((1, N), lambda i: (0, 0)),
        ],
        out_specs=pl.BlockSpec((TR, 2, 2 * W, C_out), lambda i: (i, 0, 0, 0)),
        compiler_params=pltpu.CompilerParams(
            dimension_semantics=("parallel",),
            vmem_limit_bytes=60 * 1024 * 1024,
        ),
        cost_estimate=cost,
    )(x2d, w_mat, b_mat)
    return out4d.reshape(B, 4 * L, C_out)


# speedup vs baseline: 1.0077x; 1.0077x over previous
import jax
import jax.numpy as jnp
from jax.experimental import pallas as pl
from jax.experimental.pallas import tpu as pltpu

_H, _W = 28, 28


def _upsample_kernel(x_ref, w_ref, b_ref, o_ref):
    TR, _, W2, C_out = o_ref.shape
    W = W2 // 2
    n2 = 2 * C_out
    xb = x_ref[...].astype(jnp.bfloat16)
    acc = jnp.dot(xb, w_ref[...], preferred_element_type=jnp.float32)
    acc = acc + b_ref[...]
    o_ref[:, 0, :, :] = acc[:, :n2].reshape(TR * W * 2, C_out).reshape(
        TR, W2, C_out)
    o_ref[:, 1, :, :] = acc[:, n2:].reshape(TR * W * 2, C_out).reshape(
        TR, W2, C_out)


def kernel(x, weight, bias):
    H, W = _H, _W
    B, L, C_in = x.shape
    assert L == H * W
    _, C_out, kh_, kw_ = weight.shape
    assert (kh_, kw_) == (2, 2)

    N = 4 * C_out
    BH = B * H

    x2d = x.reshape(BH * W, C_in)
    w_mat = jnp.transpose(weight, (0, 2, 3, 1)).reshape(C_in, N)
    w_mat = w_mat.astype(jnp.bfloat16)
    b_mat = jnp.tile(bias.astype(jnp.float32), 4).reshape(1, N)

    TR = 224
    if BH % TR != 0:
        TR = 8 if BH % 8 == 0 else 1
    grid = (BH // TR,)

    cost = pl.CostEstimate(
        flops=2 * BH * W * C_in * N,
        bytes_accessed=(x2d.size * 4 + w_mat.size * 2 + b_mat.size * 4
                        + BH * 4 * W * C_out * 4),
        transcendentals=0,
    )

    out4d = pl.pallas_call(
        _upsample_kernel,
        out_shape=jax.ShapeDtypeStruct((BH, 2, 2 * W, C_out), x.dtype),
        grid=grid,
        in_specs=[
            pl.BlockSpec((TR * W, C_in), lambda i: (i, 0)),
            pl.BlockSpec((C_in, N), lambda i: (0, 0)),
            pl.BlockSpec((1, N), lambda i: (0, 0)),
        ],
        out_specs=pl.BlockSpec((TR, 2, 2 * W, C_out), lambda i: (i, 0, 0, 0)),
        compiler_params=pltpu.CompilerParams(
            dimension_semantics=("arbitrary",),
            vmem_limit_bytes=60 * 1024 * 1024,
        ),
        cost_estimate=cost,
    )(x2d, w_mat, b_mat)
    return out4d.reshape(B, 4 * L, C_out)


# speedup vs baseline: 1.0128x; 1.0051x over previous
import jax
import jax.numpy as jnp
from jax.experimental import pallas as pl
from jax.experimental.pallas import tpu as pltpu

_H, _W = 28, 28


def _upsample_kernel(x_ref, w_ref, b_ref, o_ref):
    TR, _, W2, C_out = o_ref.shape
    W = W2 // 2
    n2 = 2 * C_out
    acc = jnp.dot(x_ref[...], w_ref[...], preferred_element_type=jnp.float32)
    acc = acc + b_ref[...]
    o_ref[:, 0, :, :] = acc[:, :n2].reshape(TR * W * 2, C_out).reshape(
        TR, W2, C_out)
    o_ref[:, 1, :, :] = acc[:, n2:].reshape(TR * W * 2, C_out).reshape(
        TR, W2, C_out)


def kernel(x, weight, bias):
    H, W = _H, _W
    B, L, C_in = x.shape
    assert L == H * W
    _, C_out, kh_, kw_ = weight.shape
    assert (kh_, kw_) == (2, 2)

    N = 4 * C_out
    BH = B * H

    x2d = x.reshape(BH * W, C_in)
    w_mat = jnp.transpose(weight, (0, 2, 3, 1)).reshape(C_in, N)
    b_mat = jnp.tile(bias.astype(jnp.float32), 4).reshape(1, N)

    TR = 224
    if BH % TR != 0:
        TR = 8 if BH % 8 == 0 else 1
    grid = (BH // TR,)

    cost = pl.CostEstimate(
        flops=2 * BH * W * C_in * N,
        bytes_accessed=(x2d.size * 4 + w_mat.size * 4 + b_mat.size * 4
                        + BH * 4 * W * C_out * 4),
        transcendentals=0,
    )

    out4d = pl.pallas_call(
        _upsample_kernel,
        out_shape=jax.ShapeDtypeStruct((BH, 2, 2 * W, C_out), x.dtype),
        grid=grid,
        in_specs=[
            pl.BlockSpec((TR * W, C_in), lambda i: (i, 0)),
            pl.BlockSpec((C_in, N), lambda i: (0, 0)),
            pl.BlockSpec((1, N), lambda i: (0, 0)),
        ],
        out_specs=pl.BlockSpec((TR, 2, 2 * W, C_out), lambda i: (i, 0, 0, 0)),
        compiler_params=pltpu.CompilerParams(
            dimension_semantics=("arbitrary",),
            vmem_limit_bytes=60 * 1024 * 1024,
        ),
        cost_estimate=cost,
    )(x2d, w_mat, b_mat)
    return out4d.reshape(B, 4 * L, C_out)
